# baseline (speedup 1.0000x reference)
import functools

import jax
import jax.numpy as jnp
from jax import lax
from jax.experimental import pallas as pl
from jax.experimental.pallas import tpu as pltpu
from jax.experimental.pallas import tpu_sc as plsc

N = 10000
E = 320000
D = 128
R = 20
B = 20
EPS = 1e-5

NC, NS = 2, 16
NW = NC * NS
EPW = E // NW
G = 128
CH = 80
PH = 2
CPP = CH // PH
NPAIR = CPP // 2
NPS = 632
NPAD = NPS * NS


def _transform_body(comp_ref, h_ref, v_ref, y_ref):
    r = pl.program_id(0)
    w = jnp.zeros((D, D), jnp.float32)
    for b in range(B):
        w = w + comp_ref[r, b] * v_ref[b]
    y_ref[...] = jnp.dot(h_ref[...], w, preferred_element_type=jnp.float32)


def _transform(comp, h, v):
    return pl.pallas_call(
        _transform_body,
        grid=(R,),
        in_specs=[
            pl.BlockSpec(memory_space=pltpu.SMEM),
            pl.BlockSpec((N, D), lambda r: (0, 0)),
            pl.BlockSpec((B, D, D), lambda r: (0, 0, 0)),
        ],
        out_specs=pl.BlockSpec((N, D), lambda r: (r, 0)),
        out_shape=jax.ShapeDtypeStruct((R * N, D), jnp.float32),
    )(comp, h, v)


_MESH = plsc.VectorSubcoreMesh(
    core_axis_name="c", subcore_axis_name="s", num_cores=NC, num_subcores=NS
)


@functools.partial(
    pl.kernel,
    out_type=jax.ShapeDtypeStruct((NC, NPAD, D), jnp.float32),
    mesh=_MESH,
    scratch_types=[
        pltpu.VMEM((CPP, G), jnp.int32),
        pltpu.VMEM((CPP, G), jnp.int32),
        pltpu.VMEM((G, D), jnp.float32),
        pltpu.VMEM((G, D), jnp.float32),
        pltpu.VMEM_SHARED((NPAD, D), jnp.float32),
        pltpu.SemaphoreType.DMA,
        pltpu.SemaphoreType.DMA,
    ],
)
def _scatter_accum(y_hbm, gidx_hbm, dst_hbm, zeros_hbm, out_hbm,
                   idx_v, dst_v, rows0, rows1, msg_sh, sem0, sem1):
    cid = lax.axis_index("c")
    sid = lax.axis_index("s")
    wid = sid * NC + cid

    row0 = sid * NPS
    pltpu.sync_copy(zeros_hbm.at[pl.ds(row0, NPS)], msg_sh.at[pl.ds(row0, NPS)])
    plsc.subcore_barrier()

    for p in range(PH):
        pltpu.sync_copy(gidx_hbm.at[wid, pl.ds(p * CPP, CPP)], idx_v)
        pltpu.sync_copy(dst_hbm.at[wid, pl.ds(p * CPP, CPP)], dst_v)

        pltpu.async_copy(y_hbm.at[idx_v.at[0]], rows0, sem0)

        def body(j, carry):
            a = 2 * j
            b = a + 1
            pltpu.async_copy(y_hbm.at[idx_v.at[b]], rows1, sem1)
            pltpu.make_async_copy(y_hbm.at[idx_v.at[a]], rows0, sem0).wait()
            pltpu.sync_copy(rows0, msg_sh.at[dst_v.at[a]], add=True)

            @pl.when(j < NPAIR - 1)
            def _():
                pltpu.async_copy(y_hbm.at[idx_v.at[a + 2]], rows0, sem0)

            pltpu.make_async_copy(y_hbm.at[idx_v.at[b]], rows1, sem1).wait()
            pltpu.sync_copy(rows1, msg_sh.at[dst_v.at[b]], add=True)
            return carry

        lax.fori_loop(0, NPAIR, body, 0)

    plsc.subcore_barrier()
    pltpu.sync_copy(msg_sh.at[pl.ds(row0, NPS)],
                    out_hbm.at[cid, pl.ds(row0, NPS)])


def _bn_body(msg_ref, h_ref, loop_ref, bias_ref, gamma_ref, beta_ref, o_ref,
             *, relu):
    z = (msg_ref[0, :N] + msg_ref[1, :N] + bias_ref[...]
         + jnp.dot(h_ref[...], loop_ref[...], preferred_element_type=jnp.float32))
    mu = jnp.mean(z, axis=0, keepdims=True)
    d = z - mu
    var = jnp.mean(d * d, axis=0, keepdims=True)
    o = d * lax.rsqrt(var + EPS) * gamma_ref[...] + beta_ref[...]
    if relu:
        o = jnp.maximum(o, 0.0)
    o_ref[...] = o


def _bn(msg2, h, loop_w, bias, gamma, beta, relu):
    return pl.pallas_call(
        functools.partial(_bn_body, relu=relu),
        out_shape=jax.ShapeDtypeStruct((N, D), jnp.float32),
    )(msg2, h, loop_w, bias.reshape(1, D), gamma.reshape(1, D),
      beta.reshape(1, D))


def kernel(x, edge_index, edge_type,
           V0, comp0, loop0, bias0, gamma0, beta0,
           V1, comp1, loop1, bias1, gamma1, beta1,
           V2, comp2, loop2, bias2, gamma2, beta2):
    src = edge_index[0]
    dst = edge_index[1]
    pad = CH * G - EPW
    gidx = jnp.pad((edge_type * N + src).reshape(NW, EPW),
                   ((0, 0), (0, pad))).reshape(NW, CH, G)
    dst2 = jnp.pad(dst.reshape(NW, EPW), ((0, 0), (0, pad)),
                   constant_values=N).reshape(NW, CH, G)
    zeros = jnp.zeros((NPAD, D), jnp.float32)

    params = [
        (V0, comp0, loop0, bias0, gamma0, beta0),
        (V1, comp1, loop1, bias1, gamma1, beta1),
        (V2, comp2, loop2, bias2, gamma2, beta2),
    ]
    h = x
    for i, (V, comp, loop_w, bias, gamma, beta) in enumerate(params):
        y = _transform(comp, h, V)
        msg2 = _scatter_accum(y, gidx, dst2, zeros)
        h = _bn(msg2, h, loop_w, bias, gamma, beta, relu=(i == 2))
    return h


# speedup vs baseline: 2.1159x; 2.1159x over previous
import functools

import jax
import jax.numpy as jnp
from jax import lax
from jax.experimental import pallas as pl
from jax.experimental.pallas import tpu as pltpu
from jax.experimental.pallas import tpu_sc as plsc

N = 10000
E = 320000
D = 128
R = 20
B = 20
EPS = 1e-5

NC, NS = 2, 16
NW = NC * NS
EPW = E // NW
G = 128
KC = 2
GK = KC * G
CH = 40
PH = 2
CPP = CH // PH
NPS = 632
NPAD = NPS * NS
PADN = CH * GK - EPW


def _transform_body(comp_ref, h_ref, v_ref, y_ref):
    r = pl.program_id(0)
    w = jnp.zeros((D, D), jnp.float32)
    for b in range(B):
        w = w + comp_ref[r, b] * v_ref[b]
    y_ref[...] = jnp.dot(h_ref[...], w, preferred_element_type=jnp.float32)


def _transform(comp, h, v):
    return pl.pallas_call(
        _transform_body,
        grid=(R,),
        in_specs=[
            pl.BlockSpec(memory_space=pltpu.SMEM),
            pl.BlockSpec((N, D), lambda r: (0, 0)),
            pl.BlockSpec((B, D, D), lambda r: (0, 0, 0)),
        ],
        out_specs=pl.BlockSpec((N, D), lambda r: (r, 0)),
        out_shape=jax.ShapeDtypeStruct((R * N, D), jnp.float32),
    )(comp, h, v)


_MESH = plsc.VectorSubcoreMesh(
    core_axis_name="c", subcore_axis_name="s", num_cores=NC, num_subcores=NS
)


@functools.partial(
    pl.kernel,
    out_type=jax.ShapeDtypeStruct((NC, NPAD, D), jnp.float32),
    mesh=_MESH,
    scratch_types=[
        pltpu.VMEM((CPP * GK,), jnp.int32),
        pltpu.VMEM((2 * CPP, G), jnp.int32),
        pltpu.VMEM((GK, D), jnp.float32),
        pltpu.VMEM_SHARED((NPAD, D), jnp.float32),
        pltpu.SemaphoreType.DMA,
    ],
)
def _scatter_accum(y_hbm, gidx_hbm, dst_hbm, zeros_hbm, out_hbm,
                   idx_v, dst_v, rows_v, msg_sh, sem):
    cid = lax.axis_index("c")
    sid = lax.axis_index("s")
    wid = sid * NC + cid

    row0 = sid * NPS
    pltpu.sync_copy(zeros_hbm.at[pl.ds(row0, NPS)], msg_sh.at[pl.ds(row0, NPS)])
    plsc.subcore_barrier()

    for p in range(PH):
        pltpu.sync_copy(gidx_hbm.at[pl.ds((wid * PH + p) * (CPP * GK),
                                          CPP * GK)], idx_v)
        pltpu.sync_copy(dst_hbm.at[wid, p], dst_v)

        def body(i, carry):
            off = pl.multiple_of(i * GK, GK)
            pltpu.async_copy(y_hbm.at[idx_v.at[pl.ds(off, GK)]], rows_v, sem
                             ).wait()
            pltpu.sync_copy(rows_v.at[pl.ds(0, G)],
                            msg_sh.at[dst_v.at[2 * i]], add=True)
            pltpu.sync_copy(rows_v.at[pl.ds(G, G)],
                            msg_sh.at[dst_v.at[2 * i + 1]], add=True)
            return carry

        lax.fori_loop(0, CPP, body, 0)

    plsc.subcore_barrier()
    pltpu.sync_copy(msg_sh.at[pl.ds(row0, NPS)],
                    out_hbm.at[cid, pl.ds(row0, NPS)])


def _bn_body(msg_ref, h_ref, loop_ref, bias_ref, gamma_ref, beta_ref, o_ref,
             *, relu):
    z = (msg_ref[0, :N] + msg_ref[1, :N] + bias_ref[...]
         + jnp.dot(h_ref[...], loop_ref[...], preferred_element_type=jnp.float32))
    mu = jnp.mean(z, axis=0, keepdims=True)
    d = z - mu
    var = jnp.mean(d * d, axis=0, keepdims=True)
    o = d * lax.rsqrt(var + EPS) * gamma_ref[...] + beta_ref[...]
    if relu:
        o = jnp.maximum(o, 0.0)
    o_ref[...] = o


def _bn(msg2, h, loop_w, bias, gamma, beta, relu):
    return pl.pallas_call(
        functools.partial(_bn_body, relu=relu),
        out_shape=jax.ShapeDtypeStruct((N, D), jnp.float32),
    )(msg2, h, loop_w, bias.reshape(1, D), gamma.reshape(1, D),
      beta.reshape(1, D))


def kernel(x, edge_index, edge_type,
           V0, comp0, loop0, bias0, gamma0, beta0,
           V1, comp1, loop1, bias1, gamma1, beta1,
           V2, comp2, loop2, bias2, gamma2, beta2):
    src = edge_index[0]
    dst = edge_index[1]
    k = jnp.arange(NW * PADN, dtype=jnp.int32).reshape(NW, PADN)
    gpad = k % (R * N)
    dpad = N + k % (NPAD - N)
    gidx = jnp.concatenate(
        [(edge_type * N + src).reshape(NW, EPW), gpad], axis=1
    ).reshape(NW * CH * GK)
    dst2 = jnp.concatenate(
        [dst.reshape(NW, EPW), dpad], axis=1
    ).reshape(NW, PH, 2 * CPP, G)
    zeros = jnp.zeros((NPAD, D), jnp.float32)

    params = [
        (V0, comp0, loop0, bias0, gamma0, beta0),
        (V1, comp1, loop1, bias1, gamma1, beta1),
        (V2, comp2, loop2, bias2, gamma2, beta2),
    ]
    h = x
    for i, (V, comp, loop_w, bias, gamma, beta) in enumerate(params):
        y = _transform(comp, h, V)
        msg2 = _scatter_accum(y, gidx, dst2, zeros)
        h = _bn(msg2, h, loop_w, bias, gamma, beta, relu=(i == 2))
    return h


# speedup vs baseline: 2.1362x; 1.0096x over previous
import functools

import jax
import jax.numpy as jnp
from jax import lax
from jax.experimental import pallas as pl
from jax.experimental.pallas import tpu as pltpu
from jax.experimental.pallas import tpu_sc as plsc

N = 10000
E = 320000
D = 128
R = 20
B = 20
EPS = 1e-5

NC, NS = 2, 16
NW = NC * NS
EPW = E // NW
G = 128
KC = 2
GK = KC * G
CH = 40
PH = 2
CPP = CH // PH
NPS = 632
NPAD = NPS * NS
PADN = CH * GK - EPW


def _transform_body(comp_ref, h_ref, v_ref, y_ref):
    r = pl.program_id(0)
    w = jnp.zeros((D, D), jnp.float32)
    for b in range(B):
        w = w + comp_ref[r, b] * v_ref[b]
    y_ref[...] = jnp.dot(h_ref[...], w, preferred_element_type=jnp.float32)


def _transform(comp, h, v):
    return pl.pallas_call(
        _transform_body,
        grid=(R,),
        in_specs=[
            pl.BlockSpec(memory_space=pltpu.SMEM),
            pl.BlockSpec((N, D), lambda r: (0, 0)),
            pl.BlockSpec((B, D, D), lambda r: (0, 0, 0)),
        ],
        out_specs=pl.BlockSpec((N, D), lambda r: (r, 0)),
        out_shape=jax.ShapeDtypeStruct((R * N, D), jnp.float32),
    )(comp, h, v)


_MESH = plsc.VectorSubcoreMesh(
    core_axis_name="c", subcore_axis_name="s", num_cores=NC, num_subcores=NS
)


@functools.partial(
    pl.kernel,
    out_type=jax.ShapeDtypeStruct((NC, NPAD, D), jnp.float32),
    mesh=_MESH,
    scratch_types=[
        pltpu.VMEM((CPP * GK,), jnp.int32),
        pltpu.VMEM((CPP * GK,), jnp.int32),
        pltpu.VMEM((GK, D), jnp.float32),
        pltpu.VMEM_SHARED((NPAD, D), jnp.float32),
        pltpu.SemaphoreType.DMA,
    ],
)
def _scatter_accum(y_hbm, gidx_hbm, dst_hbm, zeros_hbm, out_hbm,
                   idx_v, dst_v, rows_v, msg_sh, sem):
    cid = lax.axis_index("c")
    sid = lax.axis_index("s")
    wid = sid * NC + cid

    row0 = sid * NPS
    pltpu.sync_copy(zeros_hbm.at[pl.ds(row0, NPS)], msg_sh.at[pl.ds(row0, NPS)])
    plsc.subcore_barrier()

    for p in range(PH):
        pltpu.sync_copy(gidx_hbm.at[pl.ds((wid * PH + p) * (CPP * GK),
                                          CPP * GK)], idx_v)
        pltpu.sync_copy(dst_hbm.at[pl.ds((wid * PH + p) * (CPP * GK),
                                         CPP * GK)], dst_v)

        def body(i, carry):
            off = pl.multiple_of(i * GK, GK)
            pltpu.async_copy(y_hbm.at[idx_v.at[pl.ds(off, GK)]], rows_v, sem
                             ).wait()
            pltpu.sync_copy(rows_v, msg_sh.at[dst_v.at[pl.ds(off, GK)]],
                            add=True)
            return carry

        lax.fori_loop(0, CPP, body, 0)

    plsc.subcore_barrier()
    pltpu.sync_copy(msg_sh.at[pl.ds(row0, NPS)],
                    out_hbm.at[cid, pl.ds(row0, NPS)])


def _bn_body(msg_ref, h_ref, loop_ref, bias_ref, gamma_ref, beta_ref, o_ref,
             *, relu):
    z = (msg_ref[0, :N] + msg_ref[1, :N] + bias_ref[...]
         + jnp.dot(h_ref[...], loop_ref[...], preferred_element_type=jnp.float32))
    mu = jnp.mean(z, axis=0, keepdims=True)
    d = z - mu
    var = jnp.mean(d * d, axis=0, keepdims=True)
    o = d * lax.rsqrt(var + EPS) * gamma_ref[...] + beta_ref[...]
    if relu:
        o = jnp.maximum(o, 0.0)
    o_ref[...] = o


def _bn(msg2, h, loop_w, bias, gamma, beta, relu):
    return pl.pallas_call(
        functools.partial(_bn_body, relu=relu),
        out_shape=jax.ShapeDtypeStruct((N, D), jnp.float32),
    )(msg2, h, loop_w, bias.reshape(1, D), gamma.reshape(1, D),
      beta.reshape(1, D))


def kernel(x, edge_index, edge_type,
           V0, comp0, loop0, bias0, gamma0, beta0,
           V1, comp1, loop1, bias1, gamma1, beta1,
           V2, comp2, loop2, bias2, gamma2, beta2):
    src = edge_index[0]
    dst = edge_index[1]
    k = jnp.arange(NW * PADN, dtype=jnp.int32).reshape(NW, PADN)
    gpad = k % (R * N)
    dpad = N + k % (NPAD - N)
    gidx = jnp.concatenate(
        [(edge_type * N + src).reshape(NW, EPW), gpad], axis=1
    ).reshape(NW * CH * GK)
    dst2 = jnp.concatenate(
        [dst.reshape(NW, EPW), dpad], axis=1
    ).reshape(NW * CH * GK)
    zeros = jnp.zeros((NPAD, D), jnp.float32)

    params = [
        (V0, comp0, loop0, bias0, gamma0, beta0),
        (V1, comp1, loop1, bias1, gamma1, beta1),
        (V2, comp2, loop2, bias2, gamma2, beta2),
    ]
    h = x
    for i, (V, comp, loop_w, bias, gamma, beta) in enumerate(params):
        y = _transform(comp, h, V)
        msg2 = _scatter_accum(y, gidx, dst2, zeros)
        h = _bn(msg2, h, loop_w, bias, gamma, beta, relu=(i == 2))
    return h


# speedup vs baseline: 2.6567x; 1.2437x over previous
import functools

import jax
import jax.numpy as jnp
from jax import lax
from jax.experimental import pallas as pl
from jax.experimental.pallas import tpu as pltpu
from jax.experimental.pallas import tpu_sc as plsc

N = 10000
E = 320000
D = 128
R = 20
B = 20
EPS = 1e-5

NC, NS = 2, 16
NW = NC * NS
EPW = E // NW
G = 128
KC = 2
GK = KC * G
CH = 40
PH = 2
CPP = CH // PH
NPS = 632
NPAD = NPS * NS
PADN = CH * GK - EPW
UN = CPP * KC


def _transform_body(comp_ref, h_ref, v_ref, y_ref):
    r = pl.program_id(0)
    w = jnp.zeros((D, D), jnp.float32)
    for b in range(B):
        w = w + comp_ref[r, b] * v_ref[b]
    y_ref[...] = jnp.dot(h_ref[...], w, preferred_element_type=jnp.float32)


def _transform(comp, h, v):
    return pl.pallas_call(
        _transform_body,
        grid=(R,),
        in_specs=[
            pl.BlockSpec(memory_space=pltpu.SMEM),
            pl.BlockSpec((N, D), lambda r: (0, 0)),
            pl.BlockSpec((B, D, D), lambda r: (0, 0, 0)),
        ],
        out_specs=pl.BlockSpec((N, D), lambda r: (r, 0)),
        out_shape=jax.ShapeDtypeStruct((R * N, D), jnp.float32),
    )(comp, h, v)


_MESH = plsc.VectorSubcoreMesh(
    core_axis_name="c", subcore_axis_name="s", num_cores=NC, num_subcores=NS
)


@functools.partial(
    pl.kernel,
    out_type=jax.ShapeDtypeStruct((NC, NPAD, D), jnp.float32),
    mesh=_MESH,
    scratch_types=[
        pltpu.VMEM((CPP * GK,), jnp.int32),
        pltpu.VMEM((CPP * GK,), jnp.int32),
        pltpu.VMEM((G, D), jnp.float32),
        pltpu.VMEM((G, D), jnp.float32),
        pltpu.VMEM_SHARED((NPAD, D), jnp.float32),
        pltpu.SemaphoreType.DMA,
        pltpu.SemaphoreType.DMA,
    ],
)
def _scatter_accum(y_hbm, gidx_hbm, dst_hbm, zeros_hbm, out_hbm,
                   idx_v, dst_v, buf_a, buf_b, msg_sh, sem_a, sem_b):
    cid = lax.axis_index("c")
    sid = lax.axis_index("s")
    wid = sid * NC + cid

    row0 = sid * NPS
    pltpu.sync_copy(zeros_hbm.at[pl.ds(row0, NPS)], msg_sh.at[pl.ds(row0, NPS)])
    plsc.subcore_barrier()

    for p in range(PH):
        pltpu.sync_copy(gidx_hbm.at[pl.ds((wid * PH + p) * (CPP * GK),
                                          CPP * GK)], idx_v)
        pltpu.sync_copy(dst_hbm.at[pl.ds((wid * PH + p) * (CPP * GK),
                                         CPP * GK)], dst_v)

        def fire(u, buf, sem):
            off = pl.multiple_of(u * G, G)
            pltpu.async_copy(y_hbm.at[idx_v.at[pl.ds(off, G)]], buf, sem)

        def wait(buf, sem):
            pltpu.make_async_copy(y_hbm.at[idx_v.at[pl.ds(0, G)]], buf, sem
                                  ).wait()

        def scat(u, buf):
            off = pl.multiple_of(u * G, G)
            pltpu.sync_copy(buf, msg_sh.at[dst_v.at[pl.ds(off, G)]], add=True)

        fire(0, buf_a, sem_a)

        def body(j, carry):
            fire(2 * j + 1, buf_b, sem_b)
            wait(buf_a, sem_a)
            scat(2 * j, buf_a)
            fire(2 * j + 2, buf_a, sem_a)
            wait(buf_b, sem_b)
            scat(2 * j + 1, buf_b)
            return carry

        lax.fori_loop(0, UN // 2 - 1, body, 0)
        fire(UN - 1, buf_b, sem_b)
        wait(buf_a, sem_a)
        scat(UN - 2, buf_a)
        wait(buf_b, sem_b)
        scat(UN - 1, buf_b)

    plsc.subcore_barrier()
    pltpu.sync_copy(msg_sh.at[pl.ds(row0, NPS)],
                    out_hbm.at[cid, pl.ds(row0, NPS)])


def _bn_body(msg_ref, h_ref, loop_ref, bias_ref, gamma_ref, beta_ref, o_ref,
             *, relu):
    z = (msg_ref[0, :N] + msg_ref[1, :N] + bias_ref[...]
         + jnp.dot(h_ref[...], loop_ref[...], preferred_element_type=jnp.float32))
    mu = jnp.mean(z, axis=0, keepdims=True)
    d = z - mu
    var = jnp.mean(d * d, axis=0, keepdims=True)
    o = d * lax.rsqrt(var + EPS) * gamma_ref[...] + beta_ref[...]
    if relu:
        o = jnp.maximum(o, 0.0)
    o_ref[...] = o


def _bn(msg2, h, loop_w, bias, gamma, beta, relu):
    return pl.pallas_call(
        functools.partial(_bn_body, relu=relu),
        out_shape=jax.ShapeDtypeStruct((N, D), jnp.float32),
    )(msg2, h, loop_w, bias.reshape(1, D), gamma.reshape(1, D),
      beta.reshape(1, D))


def kernel(x, edge_index, edge_type,
           V0, comp0, loop0, bias0, gamma0, beta0,
           V1, comp1, loop1, bias1, gamma1, beta1,
           V2, comp2, loop2, bias2, gamma2, beta2):
    src = edge_index[0]
    dst = edge_index[1]
    k = jnp.arange(NW * PADN, dtype=jnp.int32).reshape(NW, PADN)
    gpad = k % (R * N)
    dpad = N + k % (NPAD - N)
    gidx = jnp.concatenate(
        [(edge_type * N + src).reshape(NW, EPW), gpad], axis=1
    ).reshape(NW * CH * GK)
    dst2 = jnp.concatenate(
        [dst.reshape(NW, EPW), dpad], axis=1
    ).reshape(NW * CH * GK)
    zeros = jnp.zeros((NPAD, D), jnp.float32)

    params = [
        (V0, comp0, loop0, bias0, gamma0, beta0),
        (V1, comp1, loop1, bias1, gamma1, beta1),
        (V2, comp2, loop2, bias2, gamma2, beta2),
    ]
    h = x
    for i, (V, comp, loop_w, bias, gamma, beta) in enumerate(params):
        y = _transform(comp, h, V)
        msg2 = _scatter_accum(y, gidx, dst2, zeros)
        h = _bn(msg2, h, loop_w, bias, gamma, beta, relu=(i == 2))
    return h


# speedup vs baseline: 2.8320x; 1.0660x over previous
import functools

import jax
import jax.numpy as jnp
from jax import lax
from jax.experimental import pallas as pl
from jax.experimental.pallas import tpu as pltpu
from jax.experimental.pallas import tpu_sc as plsc

N = 10000
E = 320000
D = 128
R = 20
B = 20
EPS = 1e-5

NC, NS = 2, 16
NW = NC * NS
EPW = E // NW
G = 80
CH = 128
PH = 2
UN = CH // PH
CPG = UN * G
NPS = 632
NPAD = NPS * NS
PADN = CH * G - EPW


def _transform_body(comp_ref, h_ref, v_ref, y_ref):
    r = pl.program_id(0)
    w = jnp.zeros((D, D), jnp.float32)
    for b in range(B):
        w = w + comp_ref[r, b] * v_ref[b]
    y_ref[...] = jnp.dot(h_ref[...], w, preferred_element_type=jnp.float32)


def _transform(comp, h, v):
    return pl.pallas_call(
        _transform_body,
        grid=(R,),
        in_specs=[
            pl.BlockSpec(memory_space=pltpu.SMEM),
            pl.BlockSpec((N, D), lambda r: (0, 0)),
            pl.BlockSpec((B, D, D), lambda r: (0, 0, 0)),
        ],
        out_specs=pl.BlockSpec((N, D), lambda r: (r, 0)),
        out_shape=jax.ShapeDtypeStruct((R * N, D), jnp.float32),
    )(comp, h, v)


_MESH = plsc.VectorSubcoreMesh(
    core_axis_name="c", subcore_axis_name="s", num_cores=NC, num_subcores=NS
)


@functools.partial(
    pl.kernel,
    out_type=jax.ShapeDtypeStruct((NC, NPAD, D), jnp.float32),
    mesh=_MESH,
    scratch_types=[
        pltpu.VMEM((CPG,), jnp.int32),
        pltpu.VMEM((CPG,), jnp.int32),
        pltpu.VMEM((G, D), jnp.float32),
        pltpu.VMEM((G, D), jnp.float32),
        pltpu.VMEM((G, D), jnp.float32),
        pltpu.VMEM_SHARED((NPAD, D), jnp.float32),
        pltpu.SemaphoreType.DMA,
        pltpu.SemaphoreType.DMA,
        pltpu.SemaphoreType.DMA,
    ],
)
def _scatter_accum(y_hbm, gidx_hbm, dst_hbm, zeros_hbm, out_hbm,
                   idx_v, dst_v, buf_a, buf_b, buf_c, msg_sh,
                   sem_a, sem_b, sem_c):
    cid = lax.axis_index("c")
    sid = lax.axis_index("s")
    wid = sid * NC + cid

    row0 = sid * NPS
    pltpu.sync_copy(zeros_hbm.at[pl.ds(row0, NPS)], msg_sh.at[pl.ds(row0, NPS)])
    plsc.subcore_barrier()

    for p in range(PH):
        pltpu.sync_copy(gidx_hbm.at[pl.ds((wid * PH + p) * CPG, CPG)], idx_v)
        pltpu.sync_copy(dst_hbm.at[pl.ds((wid * PH + p) * CPG, CPG)], dst_v)

        def fire(u, buf, sem):
            off = pl.multiple_of(u * G, G)
            pltpu.async_copy(y_hbm.at[idx_v.at[pl.ds(off, G)]], buf, sem)

        def wait(buf, sem):
            pltpu.make_async_copy(y_hbm.at[idx_v.at[pl.ds(0, G)]], buf, sem
                                  ).wait()

        def scat(u, buf):
            off = pl.multiple_of(u * G, G)
            pltpu.sync_copy(buf, msg_sh.at[dst_v.at[pl.ds(off, G)]], add=True)

        fire(0, buf_a, sem_a)
        fire(1, buf_b, sem_b)

        def body(m, carry):
            u = 3 * m
            fire(u + 2, buf_c, sem_c)
            wait(buf_a, sem_a)
            scat(u, buf_a)
            fire(u + 3, buf_a, sem_a)
            wait(buf_b, sem_b)
            scat(u + 1, buf_b)
            fire(u + 4, buf_b, sem_b)
            wait(buf_c, sem_c)
            scat(u + 2, buf_c)
            return carry

        lax.fori_loop(0, (UN - 4) // 3, body, 0)
        fire(UN - 2, buf_c, sem_c)
        wait(buf_a, sem_a)
        scat(UN - 4, buf_a)
        fire(UN - 1, buf_a, sem_a)
        wait(buf_b, sem_b)
        scat(UN - 3, buf_b)
        wait(buf_c, sem_c)
        scat(UN - 2, buf_c)
        wait(buf_a, sem_a)
        scat(UN - 1, buf_a)

    plsc.subcore_barrier()
    pltpu.sync_copy(msg_sh.at[pl.ds(row0, NPS)],
                    out_hbm.at[cid, pl.ds(row0, NPS)])


def _bn_body(msg_ref, h_ref, loop_ref, bias_ref, gamma_ref, beta_ref, o_ref,
             *, relu):
    z = (msg_ref[0, :N] + msg_ref[1, :N] + bias_ref[...]
         + jnp.dot(h_ref[...], loop_ref[...], preferred_element_type=jnp.float32))
    mu = jnp.mean(z, axis=0, keepdims=True)
    d = z - mu
    var = jnp.mean(d * d, axis=0, keepdims=True)
    o = d * lax.rsqrt(var + EPS) * gamma_ref[...] + beta_ref[...]
    if relu:
        o = jnp.maximum(o, 0.0)
    o_ref[...] = o


def _bn(msg2, h, loop_w, bias, gamma, beta, relu):
    return pl.pallas_call(
        functools.partial(_bn_body, relu=relu),
        out_shape=jax.ShapeDtypeStruct((N, D), jnp.float32),
    )(msg2, h, loop_w, bias.reshape(1, D), gamma.reshape(1, D),
      beta.reshape(1, D))


def kernel(x, edge_index, edge_type,
           V0, comp0, loop0, bias0, gamma0, beta0,
           V1, comp1, loop1, bias1, gamma1, beta1,
           V2, comp2, loop2, bias2, gamma2, beta2):
    src = edge_index[0]
    dst = edge_index[1]
    k = jnp.arange(NW * PADN, dtype=jnp.int32).reshape(NW, PADN)
    gpad = k % (R * N)
    dpad = N + k % (NPAD - N)
    gidx = jnp.concatenate(
        [(edge_type * N + src).reshape(NW, EPW), gpad], axis=1
    ).reshape(NW * CH * G)
    dst2 = jnp.concatenate(
        [dst.reshape(NW, EPW), dpad], axis=1
    ).reshape(NW * CH * G)
    zeros = jnp.zeros((NPAD, D), jnp.float32)

    params = [
        (V0, comp0, loop0, bias0, gamma0, beta0),
        (V1, comp1, loop1, bias1, gamma1, beta1),
        (V2, comp2, loop2, bias2, gamma2, beta2),
    ]
    h = x
    for i, (V, comp, loop_w, bias, gamma, beta) in enumerate(params):
        y = _transform(comp, h, V)
        msg2 = _scatter_accum(y, gidx, dst2, zeros)
        h = _bn(msg2, h, loop_w, bias, gamma, beta, relu=(i == 2))
    return h
